# baseline (speedup 1.0000x reference)
import functools

import jax
import jax.numpy as jnp
from jax import lax
from jax.experimental import pallas as pl
from jax.experimental.pallas import tpu as pltpu
from jax.experimental.pallas import tpu_sc as plsc

F32 = jnp.float32


def _egelu(y):
    return y + y * lax.erf(y)


def _layernorm(x, g, b):
    mu = jnp.mean(x, axis=-1, keepdims=True)
    d = x - mu
    var = jnp.mean(d * d, axis=-1, keepdims=True)
    return d * lax.rsqrt(var + 1e-5) * g + b


@functools.lru_cache(maxsize=None)
def _make_gather(n_rows, d, chunk=128, sup=2, nbuf=3):
    info = plsc.get_sparse_core_info()
    nw = info.num_cores * info.num_subcores
    per_w = n_rows // nw
    sup_rows = sup * chunk
    n_sup = per_w // sup_rows
    nj = n_sup // nbuf
    assert per_w == n_sup * sup_rows and n_sup % nbuf == 0
    mesh = plsc.VectorSubcoreMesh(core_axis_name="c", subcore_axis_name="s")

    @functools.partial(
        pl.kernel,
        mesh=mesh,
        out_type=jax.ShapeDtypeStruct((n_rows, d), F32),
        scratch_types=[
            pltpu.VMEM((per_w,), jnp.int32),
        ] + [pltpu.VMEM((sup_rows, d), F32)] * nbuf
          + [pltpu.SemaphoreType.DMA] * (2 * nbuf),
    )
    def gather_k(table_hbm, idx_hbm, out_hbm, idx_v, *bufsem):
        bufs = bufsem[:nbuf]
        gsems = bufsem[nbuf:2 * nbuf]
        ssems = bufsem[2 * nbuf:]
        wid = lax.axis_index("s") * info.num_cores + lax.axis_index("c")
        base = wid * per_w
        pltpu.sync_copy(idx_hbm.at[pl.ds(base, per_w)], idx_v)

        def issue_gather(sup_i, buf, sem):
            for c in range(sup):
                off = sup_i * sup_rows + c * chunk
                pltpu.async_copy(
                    table_hbm.at[idx_v.at[pl.ds(off, chunk)]],
                    buf.at[pl.ds(c * chunk, chunk)], sem)

        def drain_gather(buf, sem):
            pltpu.make_async_copy(
                out_hbm.at[pl.ds(base, sup_rows)], buf, sem).wait()

        def issue_store(sup_i, buf, sem):
            pltpu.async_copy(
                buf, out_hbm.at[pl.ds(base + sup_i * sup_rows, sup_rows)], sem)

        def drain_store(buf, sem):
            pltpu.make_async_copy(
                buf, out_hbm.at[pl.ds(base, sup_rows)], sem).wait()

        for b in range(nbuf):
            issue_gather(b, bufs[b], gsems[b])

        def body(j, carry):
            for b in range(nbuf):
                i = nbuf * j + b
                drain_gather(bufs[b], gsems[b])
                issue_store(i, bufs[b], ssems[b])

                @pl.when(j < nj - 1)
                def _():
                    drain_store(bufs[b], ssems[b])
                    issue_gather(i + nbuf, bufs[b], gsems[b])

            return carry

        lax.fori_loop(0, nj, body, 0)
        for b in range(nbuf):
            drain_store(bufs[b], ssems[b])

    return gather_k


def _proj_body(hv_ref, w_ref, out_ref):
    out_ref[...] = jnp.dot(hv_ref[...], w_ref[...], preferred_element_type=F32)


def _round1_body(tl, k, h,
                 hv_ref, he_ref, g_ref, ma_ref, mv_ref,
                 w1a_ref, w1b_ref, w2_ref, w3_ref, win_ref, wout_ref, w11c_ref,
                 b1_ref, b2_ref, b3_ref, bin_ref, bout_ref,
                 ln1g_ref, ln1b_ref, ln2g_ref, ln2b_ref,
                 hv2_ref, p2_ref):
    hv = hv_ref[...]
    a = jnp.dot(hv, w1a_ref[...], preferred_element_type=F32) + b1_ref[...]
    a_rep = jnp.broadcast_to(a[:, None, :], (tl, k, h)).reshape(tl * k, h)
    y1 = (jnp.dot(he_ref[...], w1b_ref[...], preferred_element_type=F32)
          + a_rep + g_ref[...])
    u1 = _egelu(y1)
    u2 = _egelu(jnp.dot(u1, w2_ref[...], preferred_element_type=F32)
                + b2_ref[...])
    m = jnp.dot(u2, w3_ref[...], preferred_element_type=F32) + b3_ref[...]
    m3 = m.reshape(tl, k, h) * ma_ref[...][:, :, None]
    dh = jnp.sum(m3, axis=1)
    v = _layernorm(hv + dh, ln1g_ref[...], ln1b_ref[...])
    f = jnp.dot(_egelu(jnp.dot(v, win_ref[...], preferred_element_type=F32)
                       + bin_ref[...]),
                wout_ref[...], preferred_element_type=F32) + bout_ref[...]
    v2 = _layernorm(v + f, ln2g_ref[...], ln2b_ref[...]) * mv_ref[...]
    hv2_ref[...] = v2
    p2_ref[...] = jnp.dot(v2, w11c_ref[...], preferred_element_type=F32)


def _round2_body(tl, k, h,
                 hv_ref, he_ref, g_ref,
                 wa_ref, wb_ref, w12_ref, w13_ref,
                 b11_ref, b12_ref, b13_ref, ln3g_ref, ln3b_ref,
                 out_ref):
    a = jnp.dot(hv_ref[...], wa_ref[...], preferred_element_type=F32) \
        + b11_ref[...]
    a_rep = jnp.broadcast_to(a[:, None, :], (tl, k, h)).reshape(tl * k, h)
    he = he_ref[...]
    y1 = (jnp.dot(he, wb_ref[...], preferred_element_type=F32)
          + a_rep + g_ref[...])
    u1 = _egelu(y1)
    u2 = _egelu(jnp.dot(u1, w12_ref[...], preferred_element_type=F32)
                + b12_ref[...])
    m = jnp.dot(u2, w13_ref[...], preferred_element_type=F32) + b13_ref[...]
    out_ref[...] = _layernorm(he + m, ln3g_ref[...], ln3b_ref[...])


def _full(shape):
    return pl.BlockSpec(shape, lambda i: (0,) * len(shape))


def kernel(h_V, h_E, E_idx, mask_V, mask_attend, params):
    p = params
    _, L, K, H = h_E.shape
    FF = p['Win'].shape[1]
    TL = 128
    EDGE = TL * K

    hv = h_V.reshape(L, H)
    he = h_E.reshape(L * K, H)
    idx = E_idx.reshape(L * K).astype(jnp.int32)
    ma = mask_attend.reshape(L, K)
    mv = mask_V.reshape(L, 1)

    s = 2.0 ** -0.5
    W1, W11 = p['W1'], p['W11']
    w1a, w1b, w1c = W1[:H] * s, W1[H:2 * H] * s, W1[2 * H:] * s
    w11a, w11b, w11c = W11[:H] * s, W11[H:2 * H] * s, W11[2 * H:] * s
    r1 = lambda a: a.reshape(1, -1)

    p1 = pl.pallas_call(
        _proj_body,
        out_shape=jax.ShapeDtypeStruct((L, H), F32),
    )(hv, w1c)

    half = L * K // 2
    gather_half = _make_gather(half, H)
    g1a = gather_half(p1, idx[:half])
    g1b = gather_half(p1, idx[half:])

    L2 = L // 2
    grid2 = (L2 // TL,)
    edge_spec = pl.BlockSpec((EDGE, H), lambda i: (i, 0))
    node_spec = pl.BlockSpec((TL, H), lambda i: (i, 0))

    def _k2_half(g_half, off):
        ob = off * (L2 // TL)
        e_off = pl.BlockSpec((EDGE, H), lambda i: (i + ob, 0))
        n_off = pl.BlockSpec((TL, H), lambda i: (i + ob, 0))
        return pl.pallas_call(
            functools.partial(_round1_body, TL, K, H),
            grid=grid2,
            in_specs=[
                n_off, e_off, edge_spec,
                pl.BlockSpec((TL, K), lambda i: (i + ob, 0)),
                pl.BlockSpec((TL, 1), lambda i: (i + ob, 0)),
                _full((H, H)), _full((H, H)), _full((H, H)), _full((H, H)),
                _full((H, FF)), _full((FF, H)), _full((H, H)),
                _full((1, H)), _full((1, H)), _full((1, H)),
                _full((1, FF)), _full((1, H)),
                _full((1, H)), _full((1, H)), _full((1, H)), _full((1, H)),
            ],
            out_specs=[node_spec, node_spec],
            out_shape=[jax.ShapeDtypeStruct((L2, H), F32),
                       jax.ShapeDtypeStruct((L2, H), F32)],
            compiler_params=pltpu.CompilerParams(
                dimension_semantics=("arbitrary",)),
        )(hv, he, g_half, ma, mv,
          w1a, w1b, p['W2'] * 0.5, p['W3'] * (s / 30.0), p['Win'] * s,
          p['Wout'] * s, w11c,
          r1(p['b1'] * s), r1(p['b2'] * s), r1(p['b3'] / 30.0),
          r1(p['bin'] * s), r1(p['bout']),
          r1(p['ln1_g']), r1(p['ln1_b']), r1(p['ln2_g']), r1(p['ln2_b']))

    hv2a, p2a = _k2_half(g1a, 0)
    hv2b, p2b = _k2_half(g1b, 1)
    hv2 = jnp.concatenate([hv2a, hv2b], axis=0)
    p2 = jnp.concatenate([p2a, p2b], axis=0)

    g2 = _make_gather(L * K, H)(p2, idx)

    he_out = pl.pallas_call(
        functools.partial(_round2_body, TL, K, H),
        grid=(L // TL,),
        in_specs=[
            node_spec, edge_spec, edge_spec,
            _full((H, H)), _full((H, H)), _full((H, H)), _full((H, H)),
            _full((1, H)), _full((1, H)), _full((1, H)),
            _full((1, H)), _full((1, H)),
        ],
        out_specs=edge_spec,
        out_shape=jax.ShapeDtypeStruct((L * K, H), F32),
        compiler_params=pltpu.CompilerParams(
            dimension_semantics=("arbitrary",)),
    )(hv2, he, g2,
      w11a, w11b, p['W12'] * 0.5, p['W13'] * s,
      r1(p['b11'] * s), r1(p['b12'] * s), r1(p['b13']),
      r1(p['ln3_g']), r1(p['ln3_b']))

    return (hv2.reshape(1, L, H), he_out.reshape(1, L, K, H))


# speedup vs baseline: 1.2561x; 1.2561x over previous
import functools

import jax
import jax.numpy as jnp
from jax import lax
from jax.experimental import pallas as pl
from jax.experimental.pallas import tpu as pltpu
from jax.experimental.pallas import tpu_sc as plsc

F32 = jnp.float32


def _egelu(y):
    return y + y * lax.erf(y)


def _layernorm(x, g, b):
    mu = jnp.mean(x, axis=-1, keepdims=True)
    d = x - mu
    var = jnp.mean(d * d, axis=-1, keepdims=True)
    return d * lax.rsqrt(var + 1e-5) * g + b


@functools.lru_cache(maxsize=None)
def _make_gather(n_rows, d, n_table, chunk=128, sup=2, nbuf=3):
    info = plsc.get_sparse_core_info()
    nw = info.num_cores * info.num_subcores
    ns = info.num_subcores
    per_w = n_rows // nw
    sup_rows = sup * chunk
    n_sup = per_w // sup_rows
    nj = n_sup // nbuf
    assert per_w == n_sup * sup_rows and n_sup % nbuf == 0
    t_slab = n_table // ns
    mesh = plsc.VectorSubcoreMesh(core_axis_name="c", subcore_axis_name="s")

    @functools.partial(
        pl.kernel,
        mesh=mesh,
        out_type=jax.ShapeDtypeStruct((n_rows, d), F32),
        scratch_types=[
            pltpu.VMEM((per_w,), jnp.int32),
            pltpu.VMEM_SHARED((n_table, d), F32),
        ] + [pltpu.VMEM((sup_rows, d), F32)] * nbuf
          + [pltpu.SemaphoreType.DMA] * (2 * nbuf),
    )
    def gather_k(table_hbm, idx_hbm, out_hbm, idx_v, table_s, *bufsem):
        bufs = bufsem[:nbuf]
        gsems = bufsem[nbuf:2 * nbuf]
        ssems = bufsem[2 * nbuf:]
        cid = lax.axis_index("c")
        sid = lax.axis_index("s")
        wid = sid * info.num_cores + cid
        base = wid * per_w
        pltpu.sync_copy(table_hbm.at[pl.ds(sid * t_slab, t_slab)],
                        table_s.at[pl.ds(sid * t_slab, t_slab)])
        pltpu.sync_copy(idx_hbm.at[pl.ds(base, per_w)], idx_v)
        plsc.subcore_barrier()

        def issue_gather(sup_i, buf, sem):
            for c in range(sup):
                off = sup_i * sup_rows + c * chunk
                pltpu.async_copy(
                    table_s.at[idx_v.at[pl.ds(off, chunk)]],
                    buf.at[pl.ds(c * chunk, chunk)], sem)

        def drain_gather(buf, sem):
            pltpu.make_async_copy(
                out_hbm.at[pl.ds(base, sup_rows)], buf, sem).wait()

        def issue_store(sup_i, buf, sem):
            pltpu.async_copy(
                buf, out_hbm.at[pl.ds(base + sup_i * sup_rows, sup_rows)], sem)

        def drain_store(buf, sem):
            pltpu.make_async_copy(
                buf, out_hbm.at[pl.ds(base, sup_rows)], sem).wait()

        for b in range(nbuf):
            issue_gather(b, bufs[b], gsems[b])

        def body(j, carry):
            for b in range(nbuf):
                i = nbuf * j + b
                drain_gather(bufs[b], gsems[b])
                issue_store(i, bufs[b], ssems[b])

                @pl.when(j < nj - 1)
                def _():
                    drain_store(bufs[b], ssems[b])
                    issue_gather(i + nbuf, bufs[b], gsems[b])

            return carry

        lax.fori_loop(0, nj, body, 0)
        for b in range(nbuf):
            drain_store(bufs[b], ssems[b])

    return gather_k


def _proj_body(hv_ref, w_ref, out_ref):
    out_ref[...] = jnp.dot(hv_ref[...], w_ref[...], preferred_element_type=F32)


def _round1_body(tl, k, h,
                 hv_ref, he_ref, g_ref, ma_ref, mv_ref,
                 w1a_ref, w1b_ref, w2_ref, w3_ref, win_ref, wout_ref, w11c_ref,
                 b1_ref, b2_ref, b3_ref, bin_ref, bout_ref,
                 ln1g_ref, ln1b_ref, ln2g_ref, ln2b_ref,
                 hv2_ref, p2_ref):
    hv = hv_ref[...]
    a = jnp.dot(hv, w1a_ref[...], preferred_element_type=F32) + b1_ref[...]
    a_rep = jnp.broadcast_to(a[:, None, :], (tl, k, h)).reshape(tl * k, h)
    y1 = (jnp.dot(he_ref[...], w1b_ref[...], preferred_element_type=F32)
          + a_rep + g_ref[...])
    u1 = _egelu(y1)
    u2 = _egelu(jnp.dot(u1, w2_ref[...], preferred_element_type=F32)
                + b2_ref[...])
    m = jnp.dot(u2, w3_ref[...], preferred_element_type=F32) + b3_ref[...]
    m3 = m.reshape(tl, k, h) * ma_ref[...][:, :, None]
    dh = jnp.sum(m3, axis=1)
    v = _layernorm(hv + dh, ln1g_ref[...], ln1b_ref[...])
    f = jnp.dot(_egelu(jnp.dot(v, win_ref[...], preferred_element_type=F32)
                       + bin_ref[...]),
                wout_ref[...], preferred_element_type=F32) + bout_ref[...]
    v2 = _layernorm(v + f, ln2g_ref[...], ln2b_ref[...]) * mv_ref[...]
    hv2_ref[...] = v2
    p2_ref[...] = jnp.dot(v2, w11c_ref[...], preferred_element_type=F32)


def _round2_body(tl, k, h,
                 hv_ref, he_ref, g_ref,
                 wa_ref, wb_ref, w12_ref, w13_ref,
                 b11_ref, b12_ref, b13_ref, ln3g_ref, ln3b_ref,
                 out_ref):
    a = jnp.dot(hv_ref[...], wa_ref[...], preferred_element_type=F32) \
        + b11_ref[...]
    a_rep = jnp.broadcast_to(a[:, None, :], (tl, k, h)).reshape(tl * k, h)
    he = he_ref[...]
    y1 = (jnp.dot(he, wb_ref[...], preferred_element_type=F32)
          + a_rep + g_ref[...])
    u1 = _egelu(y1)
    u2 = _egelu(jnp.dot(u1, w12_ref[...], preferred_element_type=F32)
                + b12_ref[...])
    m = jnp.dot(u2, w13_ref[...], preferred_element_type=F32) + b13_ref[...]
    out_ref[...] = _layernorm(he + m, ln3g_ref[...], ln3b_ref[...])


def _full(shape):
    return pl.BlockSpec(shape, lambda i: (0,) * len(shape))


def kernel(h_V, h_E, E_idx, mask_V, mask_attend, params):
    p = params
    _, L, K, H = h_E.shape
    FF = p['Win'].shape[1]
    TL = 128
    EDGE = TL * K

    hv = h_V.reshape(L, H)
    he = h_E.reshape(L * K, H)
    idx = E_idx.reshape(L * K).astype(jnp.int32)
    ma = mask_attend.reshape(L, K)
    mv = mask_V.reshape(L, 1)

    s = 2.0 ** -0.5
    W1, W11 = p['W1'], p['W11']
    w1a, w1b, w1c = W1[:H] * s, W1[H:2 * H] * s, W1[2 * H:] * s
    w11a, w11b, w11c = W11[:H] * s, W11[H:2 * H] * s, W11[2 * H:] * s
    r1 = lambda a: a.reshape(1, -1)

    p1 = pl.pallas_call(
        _proj_body,
        out_shape=jax.ShapeDtypeStruct((L, H), F32),
    )(hv, w1c)

    half = L * K // 2
    gather_half = _make_gather(half, H, L)
    g1a = gather_half(p1, idx[:half])
    g1b = gather_half(p1, idx[half:])

    L2 = L // 2
    grid2 = (L2 // TL,)
    edge_spec = pl.BlockSpec((EDGE, H), lambda i: (i, 0))
    node_spec = pl.BlockSpec((TL, H), lambda i: (i, 0))

    def _k2_half(g_half, off):
        ob = off * (L2 // TL)
        e_off = pl.BlockSpec((EDGE, H), lambda i: (i + ob, 0))
        n_off = pl.BlockSpec((TL, H), lambda i: (i + ob, 0))
        return pl.pallas_call(
            functools.partial(_round1_body, TL, K, H),
            grid=grid2,
            in_specs=[
                n_off, e_off, edge_spec,
                pl.BlockSpec((TL, K), lambda i: (i + ob, 0)),
                pl.BlockSpec((TL, 1), lambda i: (i + ob, 0)),
                _full((H, H)), _full((H, H)), _full((H, H)), _full((H, H)),
                _full((H, FF)), _full((FF, H)), _full((H, H)),
                _full((1, H)), _full((1, H)), _full((1, H)),
                _full((1, FF)), _full((1, H)),
                _full((1, H)), _full((1, H)), _full((1, H)), _full((1, H)),
            ],
            out_specs=[node_spec, node_spec],
            out_shape=[jax.ShapeDtypeStruct((L2, H), F32),
                       jax.ShapeDtypeStruct((L2, H), F32)],
            compiler_params=pltpu.CompilerParams(
                dimension_semantics=("arbitrary",)),
        )(hv, he, g_half, ma, mv,
          w1a, w1b, p['W2'] * 0.5, p['W3'] * (s / 30.0), p['Win'] * s,
          p['Wout'] * s, w11c,
          r1(p['b1'] * s), r1(p['b2'] * s), r1(p['b3'] / 30.0),
          r1(p['bin'] * s), r1(p['bout']),
          r1(p['ln1_g']), r1(p['ln1_b']), r1(p['ln2_g']), r1(p['ln2_b']))

    hv2a, p2a = _k2_half(g1a, 0)
    hv2b, p2b = _k2_half(g1b, 1)
    hv2 = jnp.concatenate([hv2a, hv2b], axis=0)
    p2 = jnp.concatenate([p2a, p2b], axis=0)

    g2 = _make_gather(L * K, H, L)(p2, idx)

    he_out = pl.pallas_call(
        functools.partial(_round2_body, TL, K, H),
        grid=(L // TL,),
        in_specs=[
            node_spec, edge_spec, edge_spec,
            _full((H, H)), _full((H, H)), _full((H, H)), _full((H, H)),
            _full((1, H)), _full((1, H)), _full((1, H)),
            _full((1, H)), _full((1, H)),
        ],
        out_specs=edge_spec,
        out_shape=jax.ShapeDtypeStruct((L * K, H), F32),
        compiler_params=pltpu.CompilerParams(
            dimension_semantics=("arbitrary",)),
    )(hv2, he, g2,
      w11a, w11b, p['W12'] * 0.5, p['W13'] * s,
      r1(p['b11'] * s), r1(p['b12'] * s), r1(p['b13']),
      r1(p['ln3_g']), r1(p['ln3_b']))

    return (hv2.reshape(1, L, H), he_out.reshape(1, L, K, H))


# speedup vs baseline: 1.2644x; 1.0067x over previous
import functools

import jax
import jax.numpy as jnp
from jax import lax
from jax.experimental import pallas as pl
from jax.experimental.pallas import tpu as pltpu
from jax.experimental.pallas import tpu_sc as plsc

F32 = jnp.float32
_S = 2.0 ** -0.5


def _egelu(y):
    return y + y * lax.erf(y)


def _layernorm(x, g, b):
    mu = jnp.mean(x, axis=-1, keepdims=True)
    d = x - mu
    var = jnp.mean(d * d, axis=-1, keepdims=True)
    return d * lax.rsqrt(var + 1e-5) * g + b


@functools.lru_cache(maxsize=None)
def _make_gather(n_rows, d, n_table, chunk=128, sup=2, nbuf=3):
    info = plsc.get_sparse_core_info()
    nw = info.num_cores * info.num_subcores
    ns = info.num_subcores
    per_w = n_rows // nw
    sup_rows = sup * chunk
    n_sup = per_w // sup_rows
    nj = n_sup // nbuf
    assert per_w == n_sup * sup_rows and n_sup % nbuf == 0
    t_slab = n_table // ns
    mesh = plsc.VectorSubcoreMesh(core_axis_name="c", subcore_axis_name="s")

    @functools.partial(
        pl.kernel,
        mesh=mesh,
        out_type=jax.ShapeDtypeStruct((n_rows, d), F32),
        scratch_types=[
            pltpu.VMEM((per_w,), jnp.int32),
            pltpu.VMEM_SHARED((n_table, d), F32),
        ] + [pltpu.VMEM((sup_rows, d), F32)] * nbuf
          + [pltpu.SemaphoreType.DMA] * (2 * nbuf),
    )
    def gather_k(table_hbm, idx_hbm, out_hbm, idx_v, table_s, *bufsem):
        bufs = bufsem[:nbuf]
        gsems = bufsem[nbuf:2 * nbuf]
        ssems = bufsem[2 * nbuf:]
        cid = lax.axis_index("c")
        sid = lax.axis_index("s")
        wid = sid * info.num_cores + cid
        base = wid * per_w
        pltpu.sync_copy(table_hbm.at[pl.ds(sid * t_slab, t_slab)],
                        table_s.at[pl.ds(sid * t_slab, t_slab)])
        pltpu.sync_copy(idx_hbm.at[pl.ds(base, per_w)], idx_v)
        plsc.subcore_barrier()

        def issue_gather(sup_i, buf, sem):
            for c in range(sup):
                off = sup_i * sup_rows + c * chunk
                pltpu.async_copy(
                    table_s.at[idx_v.at[pl.ds(off, chunk)]],
                    buf.at[pl.ds(c * chunk, chunk)], sem)

        def drain_gather(buf, sem):
            pltpu.make_async_copy(
                out_hbm.at[pl.ds(base, sup_rows)], buf, sem).wait()

        def issue_store(sup_i, buf, sem):
            pltpu.async_copy(
                buf, out_hbm.at[pl.ds(base + sup_i * sup_rows, sup_rows)], sem)

        def drain_store(buf, sem):
            pltpu.make_async_copy(
                buf, out_hbm.at[pl.ds(base, sup_rows)], sem).wait()

        for b in range(nbuf):
            issue_gather(b, bufs[b], gsems[b])

        def body(j, carry):
            for b in range(nbuf):
                i = nbuf * j + b
                drain_gather(bufs[b], gsems[b])
                issue_store(i, bufs[b], ssems[b])

                @pl.when(j < nj - 1)
                def _():
                    drain_store(bufs[b], ssems[b])
                    issue_gather(i + nbuf, bufs[b], gsems[b])

            return carry

        lax.fori_loop(0, nj, body, 0)
        for b in range(nbuf):
            drain_store(bufs[b], ssems[b])

    return gather_k


def _proj_body(hv_ref, w_ref, out_ref):
    out_ref[...] = jnp.dot(hv_ref[...], w_ref[...] * _S,
                           preferred_element_type=F32)


def _round1_body(tl, k, h,
                 hv_ref, he_ref, g_ref, ma_ref, mv_ref,
                 w1a_ref, w1b_ref, w2_ref, w3_ref, win_ref, wout_ref, w11c_ref,
                 b1_ref, b2_ref, b3_ref, bin_ref, bout_ref,
                 ln1g_ref, ln1b_ref, ln2g_ref, ln2b_ref,
                 hv2_ref, p2_ref):
    hv = hv_ref[...]
    a = (jnp.dot(hv, w1a_ref[...], preferred_element_type=F32)
         + b1_ref[...]) * _S
    a_rep = jnp.broadcast_to(a[:, None, :], (tl, k, h)).reshape(tl * k, h)
    y1 = (jnp.dot(he_ref[...], w1b_ref[...] * _S, preferred_element_type=F32)
          + a_rep + g_ref[...])
    u1 = _egelu(y1)
    u2 = _egelu(jnp.dot(u1, w2_ref[...] * 0.5, preferred_element_type=F32)
                + b2_ref[...] * _S)
    m = (jnp.dot(u2, w3_ref[...] * (_S / 30.0), preferred_element_type=F32)
         + b3_ref[...] * (1.0 / 30.0))
    m3 = m.reshape(tl, k, h) * ma_ref[...][:, :, None]
    dh = jnp.sum(m3, axis=1)
    v = _layernorm(hv + dh, ln1g_ref[...], ln1b_ref[...])
    f = jnp.dot(_egelu(jnp.dot(v, win_ref[...] * _S,
                               preferred_element_type=F32)
                       + bin_ref[...] * _S),
                wout_ref[...] * _S, preferred_element_type=F32) + bout_ref[...]
    v2 = _layernorm(v + f, ln2g_ref[...], ln2b_ref[...]) * mv_ref[...]
    hv2_ref[...] = v2
    p2_ref[...] = jnp.dot(v2, w11c_ref[...] * _S, preferred_element_type=F32)


def _round2_body(tl, k, h,
                 hv_ref, he_ref, g_ref,
                 wa_ref, wb_ref, w12_ref, w13_ref,
                 b11_ref, b12_ref, b13_ref, ln3g_ref, ln3b_ref,
                 out_ref):
    a = (jnp.dot(hv_ref[...], wa_ref[...], preferred_element_type=F32)
         + b11_ref[...]) * _S
    a_rep = jnp.broadcast_to(a[:, None, :], (tl, k, h)).reshape(tl * k, h)
    he = he_ref[...]
    y1 = (jnp.dot(he, wb_ref[...] * _S, preferred_element_type=F32)
          + a_rep + g_ref[...])
    u1 = _egelu(y1)
    u2 = _egelu(jnp.dot(u1, w12_ref[...] * 0.5, preferred_element_type=F32)
                + b12_ref[...] * _S)
    m = jnp.dot(u2, w13_ref[...] * _S, preferred_element_type=F32) \
        + b13_ref[...]
    out_ref[...] = _layernorm(he + m, ln3g_ref[...], ln3b_ref[...])


def _full(shape):
    return pl.BlockSpec(shape, lambda i: (0,) * len(shape))


def kernel(h_V, h_E, E_idx, mask_V, mask_attend, params):
    p = params
    _, L, K, H = h_E.shape
    FF = p['Win'].shape[1]
    TL = 128
    EDGE = TL * K

    hv = h_V.reshape(L, H)
    he = h_E.reshape(L * K, H)
    idx = E_idx.reshape(L * K).astype(jnp.int32)
    ma = mask_attend.reshape(L, K)
    mv = mask_V.reshape(L, 1)

    W1, W11 = p['W1'], p['W11']
    r1 = lambda a: a.reshape(1, -1)

    p1 = pl.pallas_call(
        _proj_body,
        grid=(1,),
        in_specs=[pl.BlockSpec((L, H), lambda i: (0, 0)),
                  pl.BlockSpec((H, H), lambda i: (2, 0))],
        out_specs=pl.BlockSpec((L, H), lambda i: (0, 0)),
        out_shape=jax.ShapeDtypeStruct((L, H), F32),
    )(hv, W1)

    half = L * K // 2
    gather_half = _make_gather(half, H, L)
    g1a = gather_half(p1, idx[:half])
    g1b = gather_half(p1, idx[half:])

    L2 = L // 2
    grid2 = (L2 // TL,)
    edge_spec = pl.BlockSpec((EDGE, H), lambda i: (i, 0))
    node_spec = pl.BlockSpec((TL, H), lambda i: (i, 0))

    def _k2_half(g_half, off):
        ob = off * (L2 // TL)
        e_off = pl.BlockSpec((EDGE, H), lambda i: (i + ob, 0))
        n_off = pl.BlockSpec((TL, H), lambda i: (i + ob, 0))
        return pl.pallas_call(
            functools.partial(_round1_body, TL, K, H),
            grid=grid2,
            in_specs=[
                n_off, e_off, edge_spec,
                pl.BlockSpec((TL, K), lambda i: (i + ob, 0)),
                pl.BlockSpec((TL, 1), lambda i: (i + ob, 0)),
                pl.BlockSpec((H, H), lambda i: (0, 0)),
                pl.BlockSpec((H, H), lambda i: (1, 0)),
                _full((H, H)), _full((H, H)),
                _full((H, FF)), _full((FF, H)),
                pl.BlockSpec((H, H), lambda i: (2, 0)),
                _full((1, H)), _full((1, H)), _full((1, H)),
                _full((1, FF)), _full((1, H)),
                _full((1, H)), _full((1, H)), _full((1, H)), _full((1, H)),
            ],
            out_specs=[node_spec, node_spec],
            out_shape=[jax.ShapeDtypeStruct((L2, H), F32),
                       jax.ShapeDtypeStruct((L2, H), F32)],
            compiler_params=pltpu.CompilerParams(
                dimension_semantics=("arbitrary",)),
        )(hv, he, g_half, ma, mv,
          W1, W1, p['W2'], p['W3'], p['Win'], p['Wout'], W11,
          r1(p['b1']), r1(p['b2']), r1(p['b3']),
          r1(p['bin']), r1(p['bout']),
          r1(p['ln1_g']), r1(p['ln1_b']), r1(p['ln2_g']), r1(p['ln2_b']))

    hv2a, p2a = _k2_half(g1a, 0)
    hv2b, p2b = _k2_half(g1b, 1)
    hv2 = jnp.concatenate([hv2a, hv2b], axis=0)
    p2 = jnp.concatenate([p2a, p2b], axis=0)

    g2 = _make_gather(L * K, H, L)(p2, idx)

    he_out = pl.pallas_call(
        functools.partial(_round2_body, TL, K, H),
        grid=(L // TL,),
        in_specs=[
            node_spec, edge_spec, edge_spec,
            pl.BlockSpec((H, H), lambda i: (0, 0)),
            pl.BlockSpec((H, H), lambda i: (1, 0)),
            _full((H, H)), _full((H, H)),
            _full((1, H)), _full((1, H)), _full((1, H)),
            _full((1, H)), _full((1, H)),
        ],
        out_specs=edge_spec,
        out_shape=jax.ShapeDtypeStruct((L * K, H), F32),
        compiler_params=pltpu.CompilerParams(
            dimension_semantics=("arbitrary",)),
    )(hv2, he, g2,
      W11, W11, p['W12'], p['W13'],
      r1(p['b11']), r1(p['b12']), r1(p['b13']),
      r1(p['ln3_g']), r1(p['ln3_b']))

    return (hv2.reshape(1, L, H), he_out.reshape(1, L, K, H))


# speedup vs baseline: 1.2849x; 1.0162x over previous
import functools

import jax
import jax.numpy as jnp
from jax import lax
from jax.experimental import pallas as pl
from jax.experimental.pallas import tpu as pltpu
from jax.experimental.pallas import tpu_sc as plsc

F32 = jnp.float32
_S = 2.0 ** -0.5


def _egelu(y):
    return y + y * lax.erf(y)


def _layernorm(x, g, b):
    mu = jnp.mean(x, axis=-1, keepdims=True)
    d = x - mu
    var = jnp.mean(d * d, axis=-1, keepdims=True)
    return d * lax.rsqrt(var + 1e-5) * g + b


@functools.lru_cache(maxsize=None)
def _make_gather(n_rows, d, n_table, chunk=128, sup=2, nbuf=3):
    info = plsc.get_sparse_core_info()
    nw = info.num_cores * info.num_subcores
    ns = info.num_subcores
    per_w = n_rows // nw
    sup_rows = sup * chunk
    n_sup = per_w // sup_rows
    nj = n_sup // nbuf
    assert per_w == n_sup * sup_rows and n_sup % nbuf == 0
    t_slab = n_table // ns
    mesh = plsc.VectorSubcoreMesh(core_axis_name="c", subcore_axis_name="s")

    @functools.partial(
        pl.kernel,
        mesh=mesh,
        out_type=jax.ShapeDtypeStruct((n_rows, d), F32),
        scratch_types=[
            pltpu.VMEM((per_w,), jnp.int32),
            pltpu.VMEM_SHARED((n_table, d), F32),
        ] + [pltpu.VMEM((sup_rows, d), F32)] * nbuf
          + [pltpu.SemaphoreType.DMA] * (2 * nbuf),
    )
    def gather_k(table_hbm, idx_hbm, out_hbm, idx_v, table_s, *bufsem):
        bufs = bufsem[:nbuf]
        gsems = bufsem[nbuf:2 * nbuf]
        ssems = bufsem[2 * nbuf:]
        cid = lax.axis_index("c")
        sid = lax.axis_index("s")
        wid = sid * info.num_cores + cid
        base = wid * per_w
        pltpu.sync_copy(table_hbm.at[pl.ds(sid * t_slab, t_slab)],
                        table_s.at[pl.ds(sid * t_slab, t_slab)])
        pltpu.sync_copy(idx_hbm.at[pl.ds(base, per_w)], idx_v)
        plsc.subcore_barrier()

        def issue_gather(sup_i, buf, sem):
            for c in range(sup):
                off = sup_i * sup_rows + c * chunk
                pltpu.async_copy(
                    table_s.at[idx_v.at[pl.ds(off, chunk)]],
                    buf.at[pl.ds(c * chunk, chunk)], sem)

        def drain_gather(buf, sem):
            pltpu.make_async_copy(
                out_hbm.at[pl.ds(base, sup_rows)], buf, sem).wait()

        def issue_store(sup_i, buf, sem):
            pltpu.async_copy(
                buf, out_hbm.at[pl.ds(base + sup_i * sup_rows, sup_rows)], sem)

        def drain_store(buf, sem):
            pltpu.make_async_copy(
                buf, out_hbm.at[pl.ds(base, sup_rows)], sem).wait()

        for b in range(nbuf):
            issue_gather(b, bufs[b], gsems[b])

        def body(j, carry):
            for b in range(nbuf):
                i = nbuf * j + b
                drain_gather(bufs[b], gsems[b])
                issue_store(i, bufs[b], ssems[b])

                @pl.when(j < nj - 1)
                def _():
                    drain_store(bufs[b], ssems[b])
                    issue_gather(i + nbuf, bufs[b], gsems[b])

            return carry

        lax.fori_loop(0, nj, body, 0)
        for b in range(nbuf):
            drain_store(bufs[b], ssems[b])

    return gather_k


def _proj_body(hv_ref, w_ref, out_ref):
    out_ref[...] = jnp.dot(hv_ref[...], w_ref[...] * _S,
                           preferred_element_type=F32)


def _round1_body(tl, k, h,
                 hv_ref, he_ref, g_ref, ma_ref, mv_ref,
                 w1a_ref, w1b_ref, w2_ref, w3_ref, win_ref, wout_ref, w11c_ref,
                 b1_ref, b2_ref, b3_ref, bin_ref, bout_ref,
                 ln1g_ref, ln1b_ref, ln2g_ref, ln2b_ref,
                 hv2_ref, p2_ref):
    hv = hv_ref[...]
    a = (jnp.dot(hv, w1a_ref[...], preferred_element_type=F32)
         + b1_ref[...]) * _S
    a_rep = jnp.broadcast_to(a[:, None, :], (tl, k, h)).reshape(tl * k, h)
    y1 = (jnp.dot(he_ref[...], w1b_ref[...] * _S, preferred_element_type=F32)
          + a_rep + g_ref[...])
    u1 = _egelu(y1)
    u2 = _egelu(jnp.dot(u1, w2_ref[...] * 0.5, preferred_element_type=F32)
                + b2_ref[...] * _S)
    m = (jnp.dot(u2, w3_ref[...] * (_S / 30.0), preferred_element_type=F32)
         + b3_ref[...] * (1.0 / 30.0))
    m3 = m.reshape(tl, k, h) * ma_ref[...][:, :, None]
    dh = jnp.sum(m3, axis=1)
    v = _layernorm(hv + dh, ln1g_ref[...], ln1b_ref[...])
    f = jnp.dot(_egelu(jnp.dot(v, win_ref[...] * _S,
                               preferred_element_type=F32)
                       + bin_ref[...] * _S),
                wout_ref[...] * _S, preferred_element_type=F32) + bout_ref[...]
    v2 = _layernorm(v + f, ln2g_ref[...], ln2b_ref[...]) * mv_ref[...]
    hv2_ref[...] = v2
    p2_ref[...] = jnp.dot(v2, w11c_ref[...] * _S, preferred_element_type=F32)


def _round2_body(tl, k, h,
                 hv_ref, he_ref, g_ref,
                 wa_ref, wb_ref, w12_ref, w13_ref,
                 b11_ref, b12_ref, b13_ref, ln3g_ref, ln3b_ref,
                 out_ref):
    a = (jnp.dot(hv_ref[...], wa_ref[...], preferred_element_type=F32)
         + b11_ref[...]) * _S
    a_rep = jnp.broadcast_to(a[:, None, :], (tl, k, h)).reshape(tl * k, h)
    he = he_ref[...]
    y1 = (jnp.dot(he, wb_ref[...] * _S, preferred_element_type=F32)
          + a_rep + g_ref[...])
    u1 = _egelu(y1)
    u2 = _egelu(jnp.dot(u1, w12_ref[...] * 0.5, preferred_element_type=F32)
                + b12_ref[...] * _S)
    m = jnp.dot(u2, w13_ref[...] * _S, preferred_element_type=F32) \
        + b13_ref[...]
    out_ref[...] = _layernorm(he + m, ln3g_ref[...], ln3b_ref[...])


def _full(shape):
    return pl.BlockSpec(shape, lambda i: (0,) * len(shape))


def kernel(h_V, h_E, E_idx, mask_V, mask_attend, params):
    p = params
    _, L, K, H = h_E.shape
    FF = p['Win'].shape[1]
    TL = 128
    EDGE = TL * K

    hv = h_V.reshape(L, H)
    he = h_E.reshape(L * K, H)
    idx = E_idx.reshape(L * K).astype(jnp.int32)
    ma = mask_attend.reshape(L, K)
    mv = mask_V.reshape(L, 1)

    W1, W11 = p['W1'], p['W11']
    r1 = lambda a: a.reshape(1, -1)

    p1 = pl.pallas_call(
        _proj_body,
        grid=(1,),
        in_specs=[pl.BlockSpec((L, H), lambda i: (0, 0)),
                  pl.BlockSpec((H, H), lambda i: (2, 0))],
        out_specs=pl.BlockSpec((L, H), lambda i: (0, 0)),
        out_shape=jax.ShapeDtypeStruct((L, H), F32),
    )(hv, W1)

    g1 = _make_gather(L * K, H, L)(p1, idx)

    L2 = L
    grid2 = (L2 // TL,)
    edge_spec = pl.BlockSpec((EDGE, H), lambda i: (i, 0))
    node_spec = pl.BlockSpec((TL, H), lambda i: (i, 0))

    def _k2_half(g_half, off):
        ob = off * (L2 // TL)
        e_off = pl.BlockSpec((EDGE, H), lambda i: (i + ob, 0))
        n_off = pl.BlockSpec((TL, H), lambda i: (i + ob, 0))
        return pl.pallas_call(
            functools.partial(_round1_body, TL, K, H),
            grid=grid2,
            in_specs=[
                n_off, e_off, e_off,
                pl.BlockSpec((TL, K), lambda i: (i + ob, 0)),
                pl.BlockSpec((TL, 1), lambda i: (i + ob, 0)),
                pl.BlockSpec((H, H), lambda i: (0, 0)),
                pl.BlockSpec((H, H), lambda i: (1, 0)),
                _full((H, H)), _full((H, H)),
                _full((H, FF)), _full((FF, H)),
                pl.BlockSpec((H, H), lambda i: (2, 0)),
                _full((1, H)), _full((1, H)), _full((1, H)),
                _full((1, FF)), _full((1, H)),
                _full((1, H)), _full((1, H)), _full((1, H)), _full((1, H)),
            ],
            out_specs=[node_spec, node_spec],
            out_shape=[jax.ShapeDtypeStruct((L2, H), F32),
                       jax.ShapeDtypeStruct((L2, H), F32)],
            compiler_params=pltpu.CompilerParams(
                dimension_semantics=("arbitrary",)),
        )(hv, he, g_half, ma, mv,
          W1, W1, p['W2'], p['W3'], p['Win'], p['Wout'], W11,
          r1(p['b1']), r1(p['b2']), r1(p['b3']),
          r1(p['bin']), r1(p['bout']),
          r1(p['ln1_g']), r1(p['ln1_b']), r1(p['ln2_g']), r1(p['ln2_b']))

    hv2, p2 = _k2_half(g1, 0)

    g2 = _make_gather(L * K, H, L)(p2, idx)

    he_out = pl.pallas_call(
        functools.partial(_round2_body, TL, K, H),
        grid=(L // TL,),
        in_specs=[
            node_spec, edge_spec, edge_spec,
            pl.BlockSpec((H, H), lambda i: (0, 0)),
            pl.BlockSpec((H, H), lambda i: (1, 0)),
            _full((H, H)), _full((H, H)),
            _full((1, H)), _full((1, H)), _full((1, H)),
            _full((1, H)), _full((1, H)),
        ],
        out_specs=edge_spec,
        out_shape=jax.ShapeDtypeStruct((L * K, H), F32),
        compiler_params=pltpu.CompilerParams(
            dimension_semantics=("arbitrary",)),
    )(hv2, he, g2,
      W11, W11, p['W12'], p['W13'],
      r1(p['b11']), r1(p['b12']), r1(p['b13']),
      r1(p['ln3_g']), r1(p['ln3_b']))

    return (hv2.reshape(1, L, H), he_out.reshape(1, L, K, H))


# speedup vs baseline: 1.3211x; 1.0282x over previous
import functools

import jax
import jax.numpy as jnp
from jax import lax
from jax.experimental import pallas as pl
from jax.experimental.pallas import tpu as pltpu
from jax.experimental.pallas import tpu_sc as plsc

F32 = jnp.float32
_S = 2.0 ** -0.5


def _egelu(y):
    return y + y * lax.erf(y)


def _layernorm(x, g, b):
    mu = jnp.mean(x, axis=-1, keepdims=True)
    d = x - mu
    var = jnp.mean(d * d, axis=-1, keepdims=True)
    return d * lax.rsqrt(var + 1e-5) * g + b


@functools.lru_cache(maxsize=None)
def _make_gather(n_rows, d, n_table, chunk=128, sup=2, nbuf=3):
    info = plsc.get_sparse_core_info()
    nw = info.num_cores * info.num_subcores
    ns = info.num_subcores
    per_w = n_rows // nw
    sup_rows = sup * chunk
    n_sup = per_w // sup_rows
    nj = n_sup // nbuf
    assert per_w == n_sup * sup_rows and n_sup % nbuf == 0
    t_slab = n_table // ns
    mesh = plsc.VectorSubcoreMesh(core_axis_name="c", subcore_axis_name="s")

    @functools.partial(
        pl.kernel,
        mesh=mesh,
        out_type=jax.ShapeDtypeStruct((n_rows, d), F32),
        scratch_types=[
            pltpu.VMEM((per_w,), jnp.int32),
            pltpu.VMEM_SHARED((n_table, d), F32),
        ] + [pltpu.VMEM((sup_rows, d), F32)] * nbuf
          + [pltpu.SemaphoreType.DMA] * (2 * nbuf),
    )
    def gather_k(table_hbm, idx_hbm, out_hbm, idx_v, table_s, *bufsem):
        bufs = bufsem[:nbuf]
        gsems = bufsem[nbuf:2 * nbuf]
        ssems = bufsem[2 * nbuf:]
        cid = lax.axis_index("c")
        sid = lax.axis_index("s")
        wid = sid * info.num_cores + cid
        base = wid * per_w
        pltpu.sync_copy(table_hbm.at[pl.ds(sid * t_slab, t_slab)],
                        table_s.at[pl.ds(sid * t_slab, t_slab)])
        pltpu.sync_copy(idx_hbm.at[pl.ds(base, per_w)], idx_v)
        plsc.subcore_barrier()

        def issue_gather(sup_i, buf, sem):
            for c in range(sup):
                off = sup_i * sup_rows + c * chunk
                pltpu.async_copy(
                    table_s.at[idx_v.at[pl.ds(off, chunk)]],
                    buf.at[pl.ds(c * chunk, chunk)], sem)

        def drain_gather(buf, sem):
            pltpu.make_async_copy(
                out_hbm.at[pl.ds(base, sup_rows)], buf, sem).wait()

        def issue_store(sup_i, buf, sem):
            pltpu.async_copy(
                buf, out_hbm.at[pl.ds(base + sup_i * sup_rows, sup_rows)], sem)

        def drain_store(buf, sem):
            pltpu.make_async_copy(
                buf, out_hbm.at[pl.ds(base, sup_rows)], sem).wait()

        for b in range(nbuf):
            issue_gather(b, bufs[b], gsems[b])

        def body(j, carry):
            for b in range(nbuf):
                i = nbuf * j + b
                drain_gather(bufs[b], gsems[b])
                issue_store(i, bufs[b], ssems[b])

                @pl.when(j < nj - 1)
                def _():
                    drain_store(bufs[b], ssems[b])
                    issue_gather(i + nbuf, bufs[b], gsems[b])

            return carry

        lax.fori_loop(0, nj, body, 0)
        for b in range(nbuf):
            drain_store(bufs[b], ssems[b])

    return gather_k


def _proj_body(hv_ref, w_ref, out_ref):
    out_ref[...] = jnp.dot(hv_ref[...], w_ref[...] * _S,
                           preferred_element_type=F32)


def _round1_body(tl, k, h,
                 hv_ref, he_ref, g_ref, ma_ref, mv_ref,
                 w1a_ref, w1b_ref, w2_ref, w3_ref, win_ref, wout_ref, w11c_ref,
                 b1_ref, b2_ref, b3_ref, bin_ref, bout_ref,
                 ln1g_ref, ln1b_ref, ln2g_ref, ln2b_ref,
                 hv2_ref, p2_ref):
    hv = hv_ref[...]
    a = (jnp.dot(hv, w1a_ref[...], preferred_element_type=F32)
         + b1_ref[...]) * _S
    a_rep = jnp.broadcast_to(a[:, None, :], (tl, k, h)).reshape(tl * k, h)
    y1 = (jnp.dot(he_ref[...], w1b_ref[...] * _S, preferred_element_type=F32)
          + a_rep + g_ref[...])
    u1 = _egelu(y1)
    u2 = _egelu(jnp.dot(u1, w2_ref[...] * 0.5, preferred_element_type=F32)
                + b2_ref[...] * _S)
    m = (jnp.dot(u2, w3_ref[...] * (_S / 30.0), preferred_element_type=F32)
         + b3_ref[...] * (1.0 / 30.0))
    m3 = m.reshape(tl, k, h) * ma_ref[...][:, :, None]
    dh = jnp.sum(m3, axis=1)
    v = _layernorm(hv + dh, ln1g_ref[...], ln1b_ref[...])
    f = jnp.dot(_egelu(jnp.dot(v, win_ref[...] * _S,
                               preferred_element_type=F32)
                       + bin_ref[...] * _S),
                wout_ref[...] * _S, preferred_element_type=F32) + bout_ref[...]
    v2 = _layernorm(v + f, ln2g_ref[...], ln2b_ref[...]) * mv_ref[...]
    hv2_ref[...] = v2
    p2_ref[...] = jnp.dot(v2, w11c_ref[...] * _S, preferred_element_type=F32)


def _round2_body(tl, k, h,
                 hv_ref, he_ref, g_ref,
                 wa_ref, wb_ref, w12_ref, w13_ref,
                 b11_ref, b12_ref, b13_ref, ln3g_ref, ln3b_ref,
                 out_ref):
    a = (jnp.dot(hv_ref[...], wa_ref[...], preferred_element_type=F32)
         + b11_ref[...]) * _S
    a_rep = jnp.broadcast_to(a[:, None, :], (tl, k, h)).reshape(tl * k, h)
    he = he_ref[...]
    y1 = (jnp.dot(he, wb_ref[...] * _S, preferred_element_type=F32)
          + a_rep + g_ref[...])
    u1 = _egelu(y1)
    u2 = _egelu(jnp.dot(u1, w12_ref[...] * 0.5, preferred_element_type=F32)
                + b12_ref[...] * _S)
    m = jnp.dot(u2, w13_ref[...] * _S, preferred_element_type=F32) \
        + b13_ref[...]
    out_ref[...] = _layernorm(he + m, ln3g_ref[...], ln3b_ref[...])


def _full(shape):
    return pl.BlockSpec(shape, lambda i: (0,) * len(shape))


def kernel(h_V, h_E, E_idx, mask_V, mask_attend, params):
    p = params
    _, L, K, H = h_E.shape
    FF = p['Win'].shape[1]
    TL = 256
    EDGE = TL * K

    hv = h_V.reshape(L, H)
    he = h_E.reshape(L * K, H)
    idx = E_idx.reshape(L * K).astype(jnp.int32)
    ma = mask_attend.reshape(L, K)
    mv = mask_V.reshape(L, 1)

    W1, W11 = p['W1'], p['W11']
    r1 = lambda a: a.reshape(1, -1)

    p1 = pl.pallas_call(
        _proj_body,
        grid=(1,),
        in_specs=[pl.BlockSpec((L, H), lambda i: (0, 0)),
                  pl.BlockSpec((H, H), lambda i: (2, 0))],
        out_specs=pl.BlockSpec((L, H), lambda i: (0, 0)),
        out_shape=jax.ShapeDtypeStruct((L, H), F32),
    )(hv, W1)

    g1 = _make_gather(L * K, H, L)(p1, idx)

    L2 = L
    grid2 = (L2 // TL,)
    edge_spec = pl.BlockSpec((EDGE, H), lambda i: (i, 0))
    node_spec = pl.BlockSpec((TL, H), lambda i: (i, 0))

    def _k2_half(g_half, off):
        ob = off * (L2 // TL)
        e_off = pl.BlockSpec((EDGE, H), lambda i: (i + ob, 0))
        n_off = pl.BlockSpec((TL, H), lambda i: (i + ob, 0))
        return pl.pallas_call(
            functools.partial(_round1_body, TL, K, H),
            grid=grid2,
            in_specs=[
                n_off, e_off, e_off,
                pl.BlockSpec((TL, K), lambda i: (i + ob, 0)),
                pl.BlockSpec((TL, 1), lambda i: (i + ob, 0)),
                pl.BlockSpec((H, H), lambda i: (0, 0)),
                pl.BlockSpec((H, H), lambda i: (1, 0)),
                _full((H, H)), _full((H, H)),
                _full((H, FF)), _full((FF, H)),
                pl.BlockSpec((H, H), lambda i: (2, 0)),
                _full((1, H)), _full((1, H)), _full((1, H)),
                _full((1, FF)), _full((1, H)),
                _full((1, H)), _full((1, H)), _full((1, H)), _full((1, H)),
            ],
            out_specs=[node_spec, node_spec],
            out_shape=[jax.ShapeDtypeStruct((L2, H), F32),
                       jax.ShapeDtypeStruct((L2, H), F32)],
            compiler_params=pltpu.CompilerParams(
                dimension_semantics=("arbitrary",)),
        )(hv, he, g_half, ma, mv,
          W1, W1, p['W2'], p['W3'], p['Win'], p['Wout'], W11,
          r1(p['b1']), r1(p['b2']), r1(p['b3']),
          r1(p['bin']), r1(p['bout']),
          r1(p['ln1_g']), r1(p['ln1_b']), r1(p['ln2_g']), r1(p['ln2_b']))

    hv2, p2 = _k2_half(g1, 0)

    g2 = _make_gather(L * K, H, L)(p2, idx)

    he_out = pl.pallas_call(
        functools.partial(_round2_body, TL, K, H),
        grid=(L // TL,),
        in_specs=[
            node_spec, edge_spec, edge_spec,
            pl.BlockSpec((H, H), lambda i: (0, 0)),
            pl.BlockSpec((H, H), lambda i: (1, 0)),
            _full((H, H)), _full((H, H)),
            _full((1, H)), _full((1, H)), _full((1, H)),
            _full((1, H)), _full((1, H)),
        ],
        out_specs=edge_spec,
        out_shape=jax.ShapeDtypeStruct((L * K, H), F32),
        compiler_params=pltpu.CompilerParams(
            dimension_semantics=("arbitrary",)),
    )(hv2, he, g2,
      W11, W11, p['W12'], p['W13'],
      r1(p['b11']), r1(p['b12']), r1(p['b13']),
      r1(p['ln3_g']), r1(p['ln3_b']))

    return (hv2.reshape(1, L, H), he_out.reshape(1, L, K, H))


# speedup vs baseline: 1.3220x; 1.0007x over previous
import functools

import jax
import jax.numpy as jnp
from jax import lax
from jax.experimental import pallas as pl
from jax.experimental.pallas import tpu as pltpu
from jax.experimental.pallas import tpu_sc as plsc

F32 = jnp.float32
_S = 2.0 ** -0.5


def _egelu(y):
    return y + y * lax.erf(y)


def _layernorm(x, g, b):
    mu = jnp.mean(x, axis=-1, keepdims=True)
    d = x - mu
    var = jnp.mean(d * d, axis=-1, keepdims=True)
    return d * lax.rsqrt(var + 1e-5) * g + b


@functools.lru_cache(maxsize=None)
def _make_gather(n_rows, d, n_table, chunk=128, sup=2, nbuf=3):
    info = plsc.get_sparse_core_info()
    nw = info.num_cores * info.num_subcores
    ns = info.num_subcores
    per_w = n_rows // nw
    sup_rows = sup * chunk
    n_sup = per_w // sup_rows
    nj = n_sup // nbuf
    assert per_w == n_sup * sup_rows and n_sup % nbuf == 0
    t_slab = n_table // ns
    mesh = plsc.VectorSubcoreMesh(core_axis_name="c", subcore_axis_name="s")

    @functools.partial(
        pl.kernel,
        mesh=mesh,
        out_type=jax.ShapeDtypeStruct((n_rows, d), F32),
        scratch_types=[
            pltpu.VMEM((per_w,), jnp.int32),
            pltpu.VMEM_SHARED((n_table, d), F32),
        ] + [pltpu.VMEM((sup_rows, d), F32)] * nbuf
          + [pltpu.SemaphoreType.DMA] * (2 * nbuf),
    )
    def gather_k(table_hbm, idx_hbm, out_hbm, idx_v, table_s, *bufsem):
        bufs = bufsem[:nbuf]
        gsems = bufsem[nbuf:2 * nbuf]
        ssems = bufsem[2 * nbuf:]
        cid = lax.axis_index("c")
        sid = lax.axis_index("s")
        wid = sid * info.num_cores + cid
        base = wid * per_w
        pltpu.sync_copy(table_hbm.at[pl.ds(sid * t_slab, t_slab)],
                        table_s.at[pl.ds(sid * t_slab, t_slab)])
        pltpu.sync_copy(idx_hbm.at[pl.ds(base, per_w)], idx_v)
        plsc.subcore_barrier()

        def issue_gather(sup_i, buf, sem):
            for c in range(sup):
                off = sup_i * sup_rows + c * chunk
                pltpu.async_copy(
                    table_s.at[idx_v.at[pl.ds(off, chunk)]],
                    buf.at[pl.ds(c * chunk, chunk)], sem)

        def drain_gather(buf, sem):
            pltpu.make_async_copy(
                out_hbm.at[pl.ds(base, sup_rows)], buf, sem).wait()

        def issue_store(sup_i, buf, sem):
            pltpu.async_copy(
                buf, out_hbm.at[pl.ds(base + sup_i * sup_rows, sup_rows)], sem)

        def drain_store(buf, sem):
            pltpu.make_async_copy(
                buf, out_hbm.at[pl.ds(base, sup_rows)], sem).wait()

        for b in range(nbuf):
            issue_gather(b, bufs[b], gsems[b])

        def body(j, carry):
            for b in range(nbuf):
                i = nbuf * j + b
                drain_gather(bufs[b], gsems[b])
                issue_store(i, bufs[b], ssems[b])

                @pl.when(j < nj - 1)
                def _():
                    drain_store(bufs[b], ssems[b])
                    issue_gather(i + nbuf, bufs[b], gsems[b])

            return carry

        lax.fori_loop(0, nj, body, 0)
        for b in range(nbuf):
            drain_store(bufs[b], ssems[b])

    return gather_k


def _proj_body(hv_ref, w_ref, out_ref):
    out_ref[...] = jnp.dot(hv_ref[...], w_ref[...] * _S,
                           preferred_element_type=F32)


def _round1_body(tl, k, h,
                 hv_ref, he_ref, g_ref, ma_ref, mv_ref,
                 w1a_ref, w1b_ref, w2_ref, w3_ref, win_ref, wout_ref, w11c_ref,
                 b1_ref, b2_ref, b3_ref, bin_ref, bout_ref,
                 ln1g_ref, ln1b_ref, ln2g_ref, ln2b_ref,
                 hv2_ref, p2_ref):
    hv = hv_ref[...]
    a = (jnp.dot(hv, w1a_ref[...], preferred_element_type=F32)
         + b1_ref[...]) * _S
    a_rep = jnp.broadcast_to(a[:, None, :], (tl, k, h)).reshape(tl * k, h)
    y1 = (jnp.dot(he_ref[...], w1b_ref[...] * _S, preferred_element_type=F32)
          + a_rep + g_ref[...])
    u1 = _egelu(y1)
    u2 = _egelu(jnp.dot(u1, w2_ref[...] * 0.5, preferred_element_type=F32)
                + b2_ref[...] * _S)
    m = (jnp.dot(u2, w3_ref[...] * (_S / 30.0), preferred_element_type=F32)
         + b3_ref[...] * (1.0 / 30.0))
    m3 = m.reshape(tl, k, h) * ma_ref[...][:, :, None]
    dh = jnp.sum(m3, axis=1)
    v = _layernorm(hv + dh, ln1g_ref[...], ln1b_ref[...])
    f = jnp.dot(_egelu(jnp.dot(v, win_ref[...] * _S,
                               preferred_element_type=F32)
                       + bin_ref[...] * _S),
                wout_ref[...] * _S, preferred_element_type=F32) + bout_ref[...]
    v2 = _layernorm(v + f, ln2g_ref[...], ln2b_ref[...]) * mv_ref[...]
    hv2_ref[...] = v2
    p2_ref[...] = jnp.dot(v2, w11c_ref[...] * _S, preferred_element_type=F32)


def _round2_body(tl, k, h,
                 hv_ref, he_ref, g_ref,
                 wa_ref, wb_ref, w12_ref, w13_ref,
                 b11_ref, b12_ref, b13_ref, ln3g_ref, ln3b_ref,
                 out_ref):
    a = (jnp.dot(hv_ref[...], wa_ref[...], preferred_element_type=F32)
         + b11_ref[...]) * _S
    a_rep = jnp.broadcast_to(a[:, None, :], (tl, k, h)).reshape(tl * k, h)
    he = he_ref[...]
    y1 = (jnp.dot(he, wb_ref[...] * _S, preferred_element_type=F32)
          + a_rep + g_ref[...])
    u1 = _egelu(y1)
    u2 = _egelu(jnp.dot(u1, w12_ref[...] * 0.5, preferred_element_type=F32)
                + b12_ref[...] * _S)
    m = jnp.dot(u2, w13_ref[...] * _S, preferred_element_type=F32) \
        + b13_ref[...]
    out_ref[...] = _layernorm(he + m, ln3g_ref[...], ln3b_ref[...])


def _full(shape):
    return pl.BlockSpec(shape, lambda i: (0,) * len(shape))


def kernel(h_V, h_E, E_idx, mask_V, mask_attend, params):
    p = params
    _, L, K, H = h_E.shape
    FF = p['Win'].shape[1]
    TL = 256
    EDGE = TL * K

    hv = h_V.reshape(L, H)
    he = h_E.reshape(L * K, H)
    idx = E_idx.reshape(L * K).astype(jnp.int32)
    ma = mask_attend.reshape(L, K)
    mv = mask_V.reshape(L, 1)

    W1, W11 = p['W1'], p['W11']
    r1 = lambda a: a.reshape(1, -1)

    p1 = pl.pallas_call(
        _proj_body,
        grid=(1,),
        in_specs=[pl.BlockSpec((L, H), lambda i: (0, 0)),
                  pl.BlockSpec((H, H), lambda i: (2, 0))],
        out_specs=pl.BlockSpec((L, H), lambda i: (0, 0)),
        out_shape=jax.ShapeDtypeStruct((L, H), F32),
    )(hv, W1)

    g1 = _make_gather(L * K, H, L)(p1, idx)

    grid = (L // TL,)
    edge_spec = pl.BlockSpec((EDGE, H), lambda i: (i, 0))
    node_spec = pl.BlockSpec((TL, H), lambda i: (i, 0))
    hv2, p2 = pl.pallas_call(
        functools.partial(_round1_body, TL, K, H),
        grid=grid,
        in_specs=[
            node_spec, edge_spec, edge_spec,
            pl.BlockSpec((TL, K), lambda i: (i, 0)),
            pl.BlockSpec((TL, 1), lambda i: (i, 0)),
            pl.BlockSpec((H, H), lambda i: (0, 0)),
            pl.BlockSpec((H, H), lambda i: (1, 0)),
            _full((H, H)), _full((H, H)),
            _full((H, FF)), _full((FF, H)),
            pl.BlockSpec((H, H), lambda i: (2, 0)),
            _full((1, H)), _full((1, H)), _full((1, H)),
            _full((1, FF)), _full((1, H)),
            _full((1, H)), _full((1, H)), _full((1, H)), _full((1, H)),
        ],
        out_specs=[node_spec, node_spec],
        out_shape=[jax.ShapeDtypeStruct((L, H), F32),
                   jax.ShapeDtypeStruct((L, H), F32)],
        compiler_params=pltpu.CompilerParams(
            dimension_semantics=("arbitrary",)),
    )(hv, he, g1, ma, mv,
      W1, W1, p['W2'], p['W3'], p['Win'], p['Wout'], W11,
      r1(p['b1']), r1(p['b2']), r1(p['b3']),
      r1(p['bin']), r1(p['bout']),
      r1(p['ln1_g']), r1(p['ln1_b']), r1(p['ln2_g']), r1(p['ln2_b']))

    g2 = _make_gather(L * K, H, L)(p2, idx)

    he_out = pl.pallas_call(
        functools.partial(_round2_body, TL, K, H),
        grid=grid,
        in_specs=[
            node_spec, edge_spec, edge_spec,
            pl.BlockSpec((H, H), lambda i: (0, 0)),
            pl.BlockSpec((H, H), lambda i: (1, 0)),
            _full((H, H)), _full((H, H)),
            _full((1, H)), _full((1, H)), _full((1, H)),
            _full((1, H)), _full((1, H)),
        ],
        out_specs=edge_spec,
        out_shape=jax.ShapeDtypeStruct((L * K, H), F32),
        compiler_params=pltpu.CompilerParams(
            dimension_semantics=("arbitrary",)),
    )(hv2, he, g2,
      W11, W11, p['W12'], p['W13'],
      r1(p['b11']), r1(p['b12']), r1(p['b13']),
      r1(p['ln3_g']), r1(p['ln3_b']))

    return (hv2.reshape(1, L, H), he_out.reshape(1, L, K, H))
